# baseline (speedup 1.0000x reference)
import jax
import jax.numpy as jnp
from jax.experimental import pallas as pl
from jax.experimental.pallas import tpu as pltpu
from jax.experimental.pallas import tpu_sc as plsc

EMBED_DIM = 64
HIST = 50
SCALE = 8.0
LANES = 16

NC, NS = 2, 16
NW = NC * NS
IPC = 2 * HIST
XPC = 2
NBUF = 2


def _sc_gather_scale(packed_table, idx, par, batch):
    num_rows = idx.shape[0]
    cpw = num_rows // NW
    pd = packed_table.shape[1]

    mesh = plsc.VectorSubcoreMesh(core_axis_name="c", subcore_axis_name="s")

    @pl.kernel(
        out_type=jax.ShapeDtypeStruct((batch, HIST, EMBED_DIM), jnp.float32),
        mesh=mesh,
        scratch_types=[
            pltpu.VMEM((cpw, IPC), jnp.int32),
            pltpu.VMEM((cpw, IPC), jnp.int32),
            pltpu.VMEM((NBUF, IPC, pd), jnp.float32),
            pltpu.VMEM((NBUF, XPC, HIST, EMBED_DIM), jnp.float32),
            pltpu.SemaphoreType.DMA((NBUF,)),
            pltpu.SemaphoreType.DMA((NBUF,)),
        ],
        compiler_params=pltpu.CompilerParams(use_tc_tiling_on_sc=True),
    )
    def k(tab_hbm, i_hbm, p_hbm, o_hbm, idx_v, par_v, in_v, out_v, gsem, wsem):
        wid = jax.lax.axis_index("s") * NC + jax.lax.axis_index("c")

        pltpu.sync_copy(i_hbm.at[pl.ds(wid * cpw, cpw)], idx_v)
        pltpu.sync_copy(p_hbm.at[pl.ds(wid * cpw, cpw)], par_v)

        def start_gather(cc, b):
            pltpu.async_copy(
                tab_hbm.at[idx_v.at[cc]], in_v.at[b], gsem.at[b]
            )

        def wait_gather(cc, b):
            pltpu.make_async_copy(
                tab_hbm.at[idx_v.at[cc]], in_v.at[b], gsem.at[b]
            ).wait()

        def scale(cc, b):
            def do_row(r, half):
                base = half * EMBED_DIM
                s, rr = divmod(r, HIST)
                for c in range(0, EMBED_DIM, LANES):
                    out_v.at[b, s, rr, pl.ds(c, LANES)][...] = (
                        in_v.at[b, r, pl.ds(base + c, LANES)][...] * SCALE
                    )

            full = (IPC // LANES) * LANES
            for r0 in range(0, full, LANES):
                p = par_v[cc, pl.ds(r0, LANES)]
                for j in range(LANES):
                    do_row(r0 + j, p[j])
            if full < IPC:
                p = par_v[cc, pl.ds(IPC - LANES, LANES)]
                for j in range(LANES - (IPC - full), LANES):
                    do_row(IPC - LANES + j, p[j])

        def write_dst(cc):
            return o_hbm.at[pl.ds((wid * cpw + cc) * XPC, XPC)]

        for b in range(NBUF):
            start_gather(b, b)
        for b in range(NBUF):
            wait_gather(b, b)
            scale(b, b)
            pltpu.async_copy(out_v.at[b], write_dst(b), wsem.at[b])
            start_gather(NBUF + b, b)

        @pl.loop(1, cpw // NBUF)
        def _(r):
            for b in range(NBUF):
                cc = r * NBUF + b
                wait_gather(cc, b)
                pltpu.make_async_copy(
                    out_v.at[b], write_dst(cc - NBUF), wsem.at[b]
                ).wait()
                scale(cc, b)
                pltpu.async_copy(out_v.at[b], write_dst(cc), wsem.at[b])

                @pl.when(cc + NBUF < cpw)
                def _():
                    start_gather(cc + NBUF, b)

        for b in range(NBUF):
            pltpu.make_async_copy(
                out_v.at[b], write_dst(cpw - NBUF + b), wsem.at[b]
            ).wait()

    return k(packed_table, idx, par)


def kernel(x, table):
    b, h = x.shape
    v, d = table.shape
    xi = x.astype(jnp.int32)
    idx = (xi >> 1).reshape(b * h // IPC, IPC)
    par = (xi & 1).reshape(b * h // IPC, IPC)
    packed_table = table.reshape(v // 2, 2 * d)
    return _sc_gather_scale(packed_table, idx, par, b)


# speedup vs baseline: 1.1952x; 1.1952x over previous
import jax
import jax.numpy as jnp
from jax.experimental import pallas as pl
from jax.experimental.pallas import tpu as pltpu
from jax.experimental.pallas import tpu_sc as plsc

EMBED_DIM = 64
HIST = 50
SCALE = 8.0
LANES = 16

NC, NS = 2, 16
NW = NC * NS
IPC = 2 * HIST
XPC = 2
NBUF = 2


def _sc_gather_scale(padded_table, idx, batch):
    num_rows = idx.shape[0]
    cpw = num_rows // NW
    pd = padded_table.shape[1]

    mesh = plsc.VectorSubcoreMesh(core_axis_name="c", subcore_axis_name="s")

    @pl.kernel(
        out_type=jax.ShapeDtypeStruct((batch, HIST, EMBED_DIM), jnp.float32),
        mesh=mesh,
        scratch_types=[
            pltpu.VMEM((cpw, IPC), jnp.int32),
            pltpu.VMEM((NBUF, IPC, pd), jnp.float32),
            pltpu.VMEM((NBUF, XPC, HIST, EMBED_DIM), jnp.float32),
            pltpu.SemaphoreType.DMA((NBUF,)),
            pltpu.SemaphoreType.DMA((NBUF,)),
        ],
        compiler_params=pltpu.CompilerParams(use_tc_tiling_on_sc=True),
    )
    def k(tab_hbm, i_hbm, o_hbm, idx_v, in_v, out_v, gsem, wsem):
        wid = jax.lax.axis_index("s") * NC + jax.lax.axis_index("c")

        pltpu.sync_copy(i_hbm.at[pl.ds(wid * cpw, cpw)], idx_v)

        def start_gather(cc, b):
            pltpu.async_copy(
                tab_hbm.at[idx_v.at[cc]], in_v.at[b], gsem.at[b]
            )

        def wait_gather(cc, b):
            pltpu.make_async_copy(
                tab_hbm.at[idx_v.at[cc]], in_v.at[b], gsem.at[b]
            ).wait()

        def scale(b):
            for s in range(XPC):
                @pl.loop(0, HIST)
                def _(rr):
                    for c in range(0, EMBED_DIM, LANES):
                        out_v.at[b, s, rr, pl.ds(c, LANES)][...] = (
                            in_v.at[b, s * HIST + rr, pl.ds(c, LANES)][...]
                            * SCALE
                        )

        def write_dst(cc):
            return o_hbm.at[pl.ds((wid * cpw + cc) * XPC, XPC)]

        for b in range(NBUF):
            start_gather(b, b)
        for b in range(NBUF):
            wait_gather(b, b)
            scale(b)
            pltpu.async_copy(out_v.at[b], write_dst(b), wsem.at[b])
            start_gather(NBUF + b, b)

        @pl.loop(1, cpw // NBUF)
        def _(r):
            for b in range(NBUF):
                cc = r * NBUF + b
                wait_gather(cc, b)
                pltpu.make_async_copy(
                    out_v.at[b], write_dst(cc - NBUF), wsem.at[b]
                ).wait()
                scale(b)
                pltpu.async_copy(out_v.at[b], write_dst(cc), wsem.at[b])

                @pl.when(cc + NBUF < cpw)
                def _():
                    start_gather(cc + NBUF, b)

        for b in range(NBUF):
            pltpu.make_async_copy(
                out_v.at[b], write_dst(cpw - NBUF + b), wsem.at[b]
            ).wait()

    return k(padded_table, idx)


def kernel(x, table):
    b, h = x.shape
    v, d = table.shape
    idx = x.astype(jnp.int32).reshape(b * h // IPC, IPC)
    padded_table = jnp.concatenate([table, table], axis=1)
    return _sc_gather_scale(padded_table, idx, b)
